# Initial kernel scaffold; baseline (speedup 1.0000x reference)
import jax
import jax.numpy as jnp
from jax.experimental import pallas as pl


def kernel(x, bins):
    raise NotImplementedError("write your pallas kernel here")


# speedup vs baseline: 6.8570x; 6.8570x over previous
import functools

import jax
import jax.numpy as jnp
import numpy as np
from jax import lax
from jax.experimental import pallas as pl
from jax.experimental.pallas import tpu as pltpu
from jax.experimental.pallas import tpu_sc as plsc

MASK = 17
STRIDE = 4
B, C, H, W = 8, 3, 257, 257
NX = (H - MASK) // STRIDE + 1
NY = (W - MASK) // STRIDE + 1
NBLK = B * C * NX
OUT_ROW = NY * MASK * MASK
TOTAL = NBLK * OUT_ROW
NW = 32

CHUNK = 17624
FULL_PER_W = 46
W31_FULL = 38
WRANGE = FULL_PER_W * CHUNK
TAIL_G0 = 31 * WRANGE + W31_FULL * CHUNK
TAIL_LEN = TOTAL - TAIL_G0
TAIL_VECS = (TAIL_LEN + 15) // 16

IDXPAD = 17696
XLEN = 4376
XHALF = 4384


def _index_table() -> np.ndarray:
    m = np.arange(IDXPAD, dtype=np.int64)
    t = m // MASK
    v = m % MASK
    u = t % MASK
    iy = t // MASK
    idx = u * W + STRIDE * iy + v
    idx[OUT_ROW:] = 0
    return idx.astype(np.int32)


_IDX = _index_table()


def _sc_body(x_hbm, idx_hbm, out_hbm, idxv, xbuf, outbuf):
    wid = lax.axis_index("s") * 2 + lax.axis_index("c")
    pltpu.sync_copy(idx_hbm, idxv)
    nfull = jnp.where(wid == 31, W31_FULL, FULL_PER_W)
    iota = lax.iota(jnp.int32, 16)

    def load_block(blk, half_off):
        bc = blk // NX
        ix = blk - bc * NX
        src0 = (bc * H + STRIDE * ix) * W
        srca = (src0 // 8) * 8

        @pl.when(blk < NBLK)
        def _():
            pltpu.sync_copy(x_hbm.at[pl.ds(srca, XLEN)],
                            xbuf.at[pl.ds(half_off, XLEN)])

        return src0 - srca

    def chunk_step(t, carry):
        @pl.when(t < nfull)
        def _():
            g0 = wid * WRANGE + t * CHUNK
            blk0 = g0 // OUT_ROW
            m0 = g0 - blk0 * OUT_ROW
            s = OUT_ROW - m0
            d0 = load_block(blk0, 0)
            d1 = load_block(blk0 + 1, XHALF)
            base_b = XHALF + d1
            nhead = s // 16
            q = s - nhead * 16

            def head_one(i, c):
                iv = idxv[pl.ds(m0 + i * 16, 16)] + d0
                outbuf[pl.ds(i * 16, 16)] = plsc.load_gather(xbuf, [iv])
                return c

            def head8(i8, c):
                for j in range(8):
                    head_one(i8 * 8 + j, 0)
                return c

            lax.fori_loop(0, nhead // 8, head8, 0)
            lax.fori_loop((nhead // 8) * 8, nhead, head_one, 0)

            ph = nhead * 16
            iva = idxv[pl.ds(m0 + ph, 16)] + d0
            ivb = iota - q + base_b
            outbuf[pl.ds(ph, 16)] = plsc.load_gather(
                xbuf, [jnp.where(iota < q, iva, ivb)])

            def tail_one(j, c):
                pos = ph + 16 + j * 16
                iv = idxv[pl.ds(pos - s, 16)] + base_b
                outbuf[pl.ds(pos, 16)] = plsc.load_gather(xbuf, [iv])
                return c

            def tail8(j8, c):
                for j in range(8):
                    tail_one(j8 * 8 + j, 0)
                return c

            ntail = (CHUNK - (ph + 16) + 15) // 16
            lax.fori_loop(0, ntail // 8, tail8, 0)
            lax.fori_loop((ntail // 8) * 8, ntail, tail_one, 0)

            pltpu.sync_copy(outbuf.at[pl.ds(0, CHUNK)],
                            out_hbm.at[pl.ds(g0, CHUNK)])

        return carry

    lax.fori_loop(0, FULL_PER_W, chunk_step, 0)

    @pl.when(wid == 31)
    def _():
        blk0 = TAIL_G0 // OUT_ROW
        m0 = TAIL_G0 - blk0 * OUT_ROW
        d0 = load_block(blk0, 0)

        def tail_vec(i, c):
            iv = idxv[pl.ds(m0 + i * 16, 16)] + d0
            outbuf[pl.ds(i * 16, 16)] = plsc.load_gather(xbuf, [iv])
            return c

        lax.fori_loop(0, TAIL_VECS, tail_vec, 0)
        pltpu.sync_copy(outbuf.at[pl.ds(0, TAIL_LEN)],
                        out_hbm.at[pl.ds(TAIL_G0, TAIL_LEN)])


def kernel(x, bins):
    del bins
    x_flat = x.reshape(B * C * H * W)
    idx = jnp.asarray(_IDX)
    mesh = plsc.VectorSubcoreMesh(core_axis_name="c", subcore_axis_name="s")
    run = functools.partial(
        pl.kernel,
        mesh=mesh,
        compiler_params=pltpu.CompilerParams(
            use_tc_tiling_on_sc=False, needs_layout_passes=False),
        out_type=jax.ShapeDtypeStruct((TOTAL,), jnp.float32),
        scratch_types=[
            pltpu.VMEM((IDXPAD,), jnp.int32),
            pltpu.VMEM((XHALF + XLEN,), jnp.float32),
            pltpu.VMEM((CHUNK + 16,), jnp.float32),
        ],
    )(_sc_body)
    out = run(x_flat, idx)
    return out.reshape(B, C, NX, NY, MASK, MASK)


# speedup vs baseline: 8.2437x; 1.2022x over previous
import functools

import jax
import jax.numpy as jnp
import numpy as np
from jax import lax
from jax.experimental import pallas as pl
from jax.experimental.pallas import tpu as pltpu
from jax.experimental.pallas import tpu_sc as plsc

MASK = 17
STRIDE = 4
B, C, H, W = 8, 3, 257, 257
NX = (H - MASK) // STRIDE + 1
NY = (W - MASK) // STRIDE + 1
NBLK = B * C * NX
OUT_ROW = NY * MASK * MASK
TOTAL = NBLK * OUT_ROW
NW = 32

CHUNK = 17624
FULL_PER_W = 46
W31_FULL = 38
WRANGE = FULL_PER_W * CHUNK
TAIL_G0 = 31 * WRANGE + W31_FULL * CHUNK
TAIL_LEN = TOTAL - TAIL_G0
TAIL_VECS = (TAIL_LEN + 15) // 16

IDXPAD = 17696
XLEN = 4376
XHALF = 4384


def _index_table() -> np.ndarray:
    m = np.arange(IDXPAD, dtype=np.int64)
    t = m // MASK
    v = m % MASK
    u = t % MASK
    iy = t // MASK
    idx = u * W + STRIDE * iy + v
    idx[OUT_ROW:] = 0
    return idx.astype(np.int32)


_IDX = _index_table()


def _sc_body(x_hbm, idx_hbm, out_hbm, idxv, xbuf, outbuf):
    wid = lax.axis_index("s") * 2 + lax.axis_index("c")
    pltpu.sync_copy(idx_hbm, idxv)
    nfull = jnp.where(wid == 31, W31_FULL, FULL_PER_W)
    iota = lax.iota(jnp.int32, 16)

    def load_block(blk, half_off):
        bc = blk // NX
        ix = blk - bc * NX
        src0 = (bc * H + STRIDE * ix) * W
        srca = (src0 // 8) * 8

        @pl.when(blk < NBLK)
        def _():
            pltpu.sync_copy(x_hbm.at[pl.ds(srca, XLEN)],
                            xbuf.at[pl.ds(half_off, XLEN)])

        return src0 - srca

    def chunk_step(t, carry):
        @pl.when(t < nfull)
        def _():
            g0 = wid * WRANGE + t * CHUNK
            blk0 = g0 // OUT_ROW
            m0 = g0 - blk0 * OUT_ROW
            s = OUT_ROW - m0
            d0 = load_block(blk0, 0)
            d1 = load_block(blk0 + 1, XHALF)
            base_b = XHALF + d1
            nhead = s // 16
            q = s - nhead * 16

            @plsc.parallel_loop(0, nhead, unroll=8)
            def _head(i):
                iv = idxv[pl.ds(m0 + i * 16, 16)] + d0
                outbuf[pl.ds(i * 16, 16)] = plsc.load_gather(xbuf, [iv])

            ph = nhead * 16
            iva = idxv[pl.ds(m0 + ph, 16)] + d0
            ivb = iota - q + base_b
            outbuf[pl.ds(ph, 16)] = plsc.load_gather(
                xbuf, [jnp.where(iota < q, iva, ivb)])

            ntail = (CHUNK - (ph + 16) + 15) // 16

            @plsc.parallel_loop(0, ntail, unroll=8)
            def _tail(j):
                pos = ph + 16 + j * 16
                iv = idxv[pl.ds(pos - s, 16)] + base_b
                outbuf[pl.ds(pos, 16)] = plsc.load_gather(xbuf, [iv])

            pltpu.sync_copy(outbuf.at[pl.ds(0, CHUNK)],
                            out_hbm.at[pl.ds(g0, CHUNK)])

        return carry

    lax.fori_loop(0, FULL_PER_W, chunk_step, 0)

    @pl.when(wid == 31)
    def _():
        blk0 = TAIL_G0 // OUT_ROW
        m0 = TAIL_G0 - blk0 * OUT_ROW
        d0 = load_block(blk0, 0)

        @plsc.parallel_loop(0, TAIL_VECS, unroll=8)
        def _tail_vec(i):
            iv = idxv[pl.ds(m0 + i * 16, 16)] + d0
            outbuf[pl.ds(i * 16, 16)] = plsc.load_gather(xbuf, [iv])
        pltpu.sync_copy(outbuf.at[pl.ds(0, TAIL_LEN)],
                        out_hbm.at[pl.ds(TAIL_G0, TAIL_LEN)])


def kernel(x, bins):
    del bins
    x_flat = x.reshape(B * C * H * W)
    idx = jnp.asarray(_IDX)
    mesh = plsc.VectorSubcoreMesh(core_axis_name="c", subcore_axis_name="s")
    run = functools.partial(
        pl.kernel,
        mesh=mesh,
        compiler_params=pltpu.CompilerParams(
            use_tc_tiling_on_sc=False, needs_layout_passes=False),
        out_type=jax.ShapeDtypeStruct((TOTAL,), jnp.float32),
        scratch_types=[
            pltpu.VMEM((IDXPAD,), jnp.int32),
            pltpu.VMEM((XHALF + XLEN,), jnp.float32),
            pltpu.VMEM((CHUNK + 16,), jnp.float32),
        ],
    )(_sc_body)
    out = run(x_flat, idx)
    return out.reshape(B, C, NX, NY, MASK, MASK)


# speedup vs baseline: 37.9071x; 4.5983x over previous
import functools

import jax
import jax.numpy as jnp
from jax import lax
from jax.experimental import pallas as pl
from jax.experimental.pallas import tpu as pltpu
from jax.experimental.pallas import tpu_sc as plsc

MASK = 17
STRIDE = 4
B, C, H, W = 8, 3, 257, 257
NX = (H - MASK) // STRIDE + 1
NY = (W - MASK) // STRIDE + 1
NW = 32
NITEMS = C * NX
T_STEPS = (NITEMS + NW - 1) // NW
XSEG = MASK * W + 7
XSTEP = XSEG + 8
IY0S = (0, 16, 32, 45)


def _sc_body(x_hbm, out_hbm, xs, ob):
    wid = lax.axis_index("s") * 2 + lax.axis_index("c")
    v4 = lax.iota(jnp.int32, 16) * 4

    def item_step(t, carry):
        it = wid + NW * t

        @pl.when(it < NITEMS)
        def _():
            c = it // NX
            ix = it - c * NX
            bd = []
            for b in range(B):
                src0 = ((b * C + c) * H + STRIDE * ix) * W
                srca = (src0 // 8) * 8
                pltpu.sync_copy(x_hbm.at[pl.ds(srca, XSEG)],
                                xs.at[pl.ds(b * XSTEP, XSEG)])
                bd.append(b * XSTEP + (src0 - srca))

            def u_step(u, carry2):
                ru = u * W
                for v in range(MASK):
                    for b in range(B):
                        for iy0 in IY0S:
                            iv = v4 + (bd[b] + ru + (v + STRIDE * iy0))
                            vals = plsc.load_gather(xs, [iv])
                            ob[v, b, pl.ds(iy0, 16)] = vals
                pltpu.sync_copy(ob, out_hbm.at[c, ix, u])
                return carry2

            lax.fori_loop(0, MASK, u_step, 0)

        return carry

    lax.fori_loop(0, T_STEPS, item_step, 0)


def kernel(x, bins):
    del bins
    x_flat = x.reshape(B * C * H * W)
    mesh = plsc.VectorSubcoreMesh(core_axis_name="c", subcore_axis_name="s")
    run = functools.partial(
        pl.kernel,
        mesh=mesh,
        compiler_params=pltpu.CompilerParams(needs_layout_passes=False),
        out_type=jax.ShapeDtypeStruct((C, NX, MASK, MASK, B, NY), jnp.float32),
        scratch_types=[
            pltpu.VMEM((B * XSTEP,), jnp.float32),
            pltpu.VMEM((MASK, B, NY), jnp.float32),
        ],
    )(_sc_body)
    out_phys = run(x_flat)
    return jnp.transpose(out_phys, (4, 0, 1, 5, 2, 3))


# speedup vs baseline: 45.6217x; 1.2035x over previous
import functools

import jax
import jax.numpy as jnp
from jax import lax
from jax.experimental import pallas as pl
from jax.experimental.pallas import tpu as pltpu
from jax.experimental.pallas import tpu_sc as plsc

MASK = 17
STRIDE = 4
B, C, H, W = 8, 3, 257, 257
NX = (H - MASK) // STRIDE + 1
NY = (W - MASK) // STRIDE + 1
NW = 32
NITEMS = C * NX
T_STEPS = (NITEMS + NW - 1) // NW
XSEG = MASK * W + 7
XSTEP = XSEG + 8
IY0S = (0, 16, 32, 45)
QP = 68
RSTEP = 4 * QP


def _sc_body(x_hbm, out_hbm, xs, xr, ob):
    wid = lax.axis_index("s") * 2 + lax.axis_index("c")
    lanes = lax.iota(jnp.int32, 16)
    idxw0 = (lanes % 4) * QP + lanes // 4
    c1 = W - 16
    idxw1 = ((lanes + c1) % 4) * QP + (lanes + c1) // 4 - (c1 // 4)

    def item_step(t, carry):
        it = wid + NW * t

        @pl.when(it < NITEMS)
        def _():
            c = it // NX
            ix = it - c * NX
            bd = []
            for b in range(B):
                src0 = ((b * C + c) * H + STRIDE * ix) * W
                srca = (src0 // 8) * 8
                pltpu.sync_copy(x_hbm.at[pl.ds(srca, XSEG)],
                                xs.at[pl.ds(b * XSTEP, XSEG)])
                bd.append(b * XSTEP + (src0 - srca))

            @plsc.parallel_loop(0, MASK, unroll=1)
            def _swz(u):
                for b in range(B):
                    rbase = bd[b] + u * W
                    wbase = (b * MASK + u) * RSTEP
                    for k in range(16):
                        vals = xs[pl.ds(rbase + 16 * k, 16)]
                        plsc.store_scatter(xr, [idxw0 + (wbase + 4 * k)], vals)
                    vals = xs[pl.ds(rbase + c1, 16)]
                    plsc.store_scatter(xr, [idxw1 + (wbase + c1 // 4)], vals)

            def u_step(u, carry2):
                ur = u * RSTEP
                for v in range(MASK):
                    for b in range(B):
                        vb = b * MASK * RSTEP + (v % 4) * QP + v // 4
                        for iy0 in IY0S:
                            ob[v, b, pl.ds(iy0, 16)] = (
                                xr[pl.ds(ur + vb + iy0, 16)])
                pltpu.sync_copy(ob, out_hbm.at[c, ix, u])
                return carry2

            lax.fori_loop(0, MASK, u_step, 0)

        return carry

    lax.fori_loop(0, T_STEPS, item_step, 0)


def kernel(x, bins):
    del bins
    x_flat = x.reshape(B * C * H * W)
    mesh = plsc.VectorSubcoreMesh(core_axis_name="c", subcore_axis_name="s")
    run = functools.partial(
        pl.kernel,
        mesh=mesh,
        compiler_params=pltpu.CompilerParams(needs_layout_passes=False),
        out_type=jax.ShapeDtypeStruct((C, NX, MASK, MASK, B, NY), jnp.float32),
        scratch_types=[
            pltpu.VMEM((B * XSTEP,), jnp.float32),
            pltpu.VMEM((B * MASK * RSTEP,), jnp.float32),
            pltpu.VMEM((MASK, B, NY), jnp.float32),
        ],
    )(_sc_body)
    out_phys = run(x_flat)
    return jnp.transpose(out_phys, (4, 0, 1, 5, 2, 3))


# speedup vs baseline: 55.3502x; 1.2132x over previous
import functools

import jax
import jax.numpy as jnp
from jax import lax
from jax.experimental import pallas as pl
from jax.experimental.pallas import tpu as pltpu
from jax.experimental.pallas import tpu_sc as plsc

MASK = 17
STRIDE = 4
B, C, H, W = 8, 3, 257, 257
NX = (H - MASK) // STRIDE + 1
NY = (W - MASK) // STRIDE + 1
NW = 32
NITEMS = C * NX
T_STEPS = (NITEMS + NW - 1) // NW
XSEG = MASK * W + 7
XSTEP = XSEG + 8
IY0S = (0, 16, 32, 45)
QP = 68
RSTEP = 4 * QP


def _sc_body(x_hbm, out_hbm, xs, xr, ob, sem0, sem1):
    wid = lax.axis_index("s") * 2 + lax.axis_index("c")
    lanes = lax.iota(jnp.int32, 16)
    idxw0 = (lanes % 4) * QP + lanes // 4
    c1 = W - 16
    idxw1 = ((lanes + c1) % 4) * QP + (lanes + c1) // 4 - (c1 // 4)

    def item_step(t, carry):
        it = wid + NW * t

        @pl.when(it < NITEMS)
        def _():
            c = it // NX
            ix = it - c * NX
            bd = []
            for b in range(B):
                src0 = ((b * C + c) * H + STRIDE * ix) * W
                srca = (src0 // 8) * 8
                pltpu.sync_copy(x_hbm.at[pl.ds(srca, XSEG)],
                                xs.at[pl.ds(b * XSTEP, XSEG)])
                bd.append(b * XSTEP + (src0 - srca))

            @plsc.parallel_loop(0, MASK, unroll=1)
            def _swz(u):
                for b in range(B):
                    rbase = bd[b] + u * W
                    wbase = (b * MASK + u) * RSTEP
                    for k in range(16):
                        vals = xs[pl.ds(rbase + 16 * k, 16)]
                        plsc.store_scatter(xr, [idxw0 + (wbase + 4 * k)], vals)
                    vals = xs[pl.ds(rbase + c1, 16)]
                    plsc.store_scatter(xr, [idxw1 + (wbase + c1 // 4)], vals)

            def u_step(u, carry2):
                par = u % 2
                dst = out_hbm.at[c, ix, u]

                @pl.when(u >= 2)
                def _():
                    @pl.when(par == 0)
                    def _():
                        pltpu.make_async_copy(ob.at[0], dst, sem0).wait()

                    @pl.when(par == 1)
                    def _():
                        pltpu.make_async_copy(ob.at[1], dst, sem1).wait()

                ur = u * RSTEP
                for v in range(MASK):
                    for b in range(B):
                        vb = b * MASK * RSTEP + (v % 4) * QP + v // 4
                        for iy0 in IY0S:
                            ob[par, v, b, pl.ds(iy0, 16)] = (
                                xr[pl.ds(ur + vb + iy0, 16)])

                @pl.when(par == 0)
                def _():
                    pltpu.async_copy(ob.at[0], dst, sem0)

                @pl.when(par == 1)
                def _():
                    pltpu.async_copy(ob.at[1], dst, sem1)

                return carry2

            lax.fori_loop(0, MASK, u_step, 0)
            pltpu.make_async_copy(ob.at[0], out_hbm.at[c, ix, 16], sem0).wait()
            pltpu.make_async_copy(ob.at[1], out_hbm.at[c, ix, 15], sem1).wait()

        return carry

    lax.fori_loop(0, T_STEPS, item_step, 0)


def kernel(x, bins):
    del bins
    x_flat = x.reshape(B * C * H * W)
    mesh = plsc.VectorSubcoreMesh(core_axis_name="c", subcore_axis_name="s")
    run = functools.partial(
        pl.kernel,
        mesh=mesh,
        compiler_params=pltpu.CompilerParams(needs_layout_passes=False),
        out_type=jax.ShapeDtypeStruct((C, NX, MASK, MASK, B, NY), jnp.float32),
        scratch_types=[
            pltpu.VMEM((B * XSTEP,), jnp.float32),
            pltpu.VMEM((B * MASK * RSTEP,), jnp.float32),
            pltpu.VMEM((2, MASK, B, NY), jnp.float32),
            pltpu.SemaphoreType.DMA,
            pltpu.SemaphoreType.DMA,
        ],
    )(_sc_body)
    out_phys = run(x_flat)
    return jnp.transpose(out_phys, (4, 0, 1, 5, 2, 3))


# speedup vs baseline: 86.5581x; 1.5638x over previous
import functools

import jax
import jax.numpy as jnp
from jax import lax
from jax.experimental import pallas as pl
from jax.experimental.pallas import tpu as pltpu
from jax.experimental.pallas import tpu_sc as plsc

MASK = 17
STRIDE = 4
B, C, H, W = 8, 3, 257, 257
NX = (H - MASK) // STRIDE + 1
NY = (W - MASK) // STRIDE + 1
NW = 32
NITEMS = C * NX
T_STEPS = (NITEMS + NW - 1) // NW
XSEG = MASK * W + 7
XSTEP = XSEG + 8
IY0S = (0, 16, 32, 45)
QP = 68
RSTEP = 4 * QP


def _sc_body(x_hbm, out_hbm, xs, xr, ob, sem0, sem1):
    wid = lax.axis_index("s") * 2 + lax.axis_index("c")
    lanes = lax.iota(jnp.int32, 16)
    idxw0 = (lanes % 4) * QP + lanes // 4
    c1 = W - 16
    idxw1 = ((lanes + c1) % 4) * QP + (lanes + c1) // 4 - (c1 // 4)

    def item_step(t, carry):
        it = wid + NW * t

        @pl.when(it < NITEMS)
        def _():
            c = it // NX
            ix = it - c * NX
            bd = []
            for b in range(B):
                src0 = ((b * C + c) * H + STRIDE * ix) * W
                srca = (src0 // 8) * 8
                pltpu.sync_copy(x_hbm.at[pl.ds(srca, XSEG)],
                                xs.at[pl.ds(b * XSTEP, XSEG)])
                bd.append(b * XSTEP + (src0 - srca))

            @plsc.parallel_loop(0, MASK, unroll=2)
            def _swz(u):
                for b in range(B):
                    rbase = bd[b] + u * W
                    wbase = (b * MASK + u) * RSTEP
                    for k in range(16):
                        vals = xs[pl.ds(rbase + 16 * k, 16)]
                        plsc.store_scatter(xr, [idxw0 + (wbase + 4 * k)], vals)
                    vals = xs[pl.ds(rbase + c1, 16)]
                    plsc.store_scatter(xr, [idxw1 + (wbase + c1 // 4)], vals)

            def u_step(u, carry2):
                par = u % 2
                dst = out_hbm.at[c, ix, u]

                @pl.when(u >= 2)
                def _():
                    @pl.when(par == 0)
                    def _():
                        pltpu.make_async_copy(ob.at[0], dst, sem0).wait()

                    @pl.when(par == 1)
                    def _():
                        pltpu.make_async_copy(ob.at[1], dst, sem1).wait()

                ur = u * RSTEP

                @plsc.parallel_loop(0, MASK, unroll=2)
                def _build(v):
                    vq = (v % 4) * QP + v // 4
                    for b in range(B):
                        vb = b * MASK * RSTEP + vq
                        for iy0 in IY0S:
                            ob[par, v, b, pl.ds(iy0, 16)] = (
                                xr[pl.ds(ur + vb + iy0, 16)])

                @pl.when(par == 0)
                def _():
                    pltpu.async_copy(ob.at[0], dst, sem0)

                @pl.when(par == 1)
                def _():
                    pltpu.async_copy(ob.at[1], dst, sem1)

                return carry2

            lax.fori_loop(0, MASK, u_step, 0)
            pltpu.make_async_copy(ob.at[0], out_hbm.at[c, ix, 16], sem0).wait()
            pltpu.make_async_copy(ob.at[1], out_hbm.at[c, ix, 15], sem1).wait()

        return carry

    lax.fori_loop(0, T_STEPS, item_step, 0)


def kernel(x, bins):
    del bins
    x_flat = x.reshape(B * C * H * W)
    mesh = plsc.VectorSubcoreMesh(core_axis_name="c", subcore_axis_name="s")
    run = functools.partial(
        pl.kernel,
        mesh=mesh,
        compiler_params=pltpu.CompilerParams(needs_layout_passes=False),
        out_type=jax.ShapeDtypeStruct((C, NX, MASK, MASK, B, NY), jnp.float32),
        scratch_types=[
            pltpu.VMEM((B * XSTEP,), jnp.float32),
            pltpu.VMEM((B * MASK * RSTEP,), jnp.float32),
            pltpu.VMEM((2, MASK, B, NY), jnp.float32),
            pltpu.SemaphoreType.DMA,
            pltpu.SemaphoreType.DMA,
        ],
    )(_sc_body)
    out_phys = run(x_flat)
    return jnp.transpose(out_phys, (4, 0, 1, 5, 2, 3))
